# baseline (speedup 1.0000x reference)
import functools

import jax
import jax.numpy as jnp
from jax import lax
from jax.experimental import pallas as pl
from jax.experimental.pallas import tpu as pltpu
from jax.experimental.pallas import tpu_sc as plsc

NUM_USERS = 100000
NUM_ITEMS = 100000
EMB = 64
N_FEAT = 4
FEAT_VOCAB = 1000
FEAT_DIM = 32
LATENT = 64
B = 16384

NC, NS, L = 2, 16, 16
NW = NC * NS
BPW = B // NW
CHUNK = 128
NCHUNK = BPW // CHUNK
GROUPS = CHUNK // L

_SET_TYPES = [
    pltpu.VMEM((CHUNK,), jnp.int32),
    pltpu.VMEM((CHUNK,), jnp.int32),
    pltpu.VMEM((N_FEAT, CHUNK), jnp.int32),
    pltpu.VMEM((CHUNK, EMB), jnp.float32),
    pltpu.VMEM((CHUNK, LATENT), jnp.float32),
    pltpu.VMEM((CHUNK, EMB), jnp.float32),
    pltpu.VMEM((CHUNK, LATENT), jnp.float32),
    pltpu.VMEM((N_FEAT, CHUNK, FEAT_DIM), jnp.float32),
    pltpu.VMEM((N_FEAT, CHUNK, FEAT_DIM), jnp.float32),
    pltpu.VMEM((CHUNK,), jnp.float32),
    pltpu.VMEM((CHUNK,), jnp.float32),
]
_NSET = len(_SET_TYPES)


def _sc_body(uid_h, iid_h, uft_h, ue_h, ub_h, mf_h, ib_h, fu_h, fi_h,
             mean_h, va_h, lat_h, part_h, bias_h, *scratch):
    setA = scratch[:_NSET]
    mean_v, part_v, biasout_v, sem0, sem1 = scratch[_NSET:]
    bufsets = (setA,)
    sems = ((sem0, sem1),)

    wid = lax.axis_index("s") * NC + lax.axis_index("c")
    base = wid * BPW

    pltpu.sync_copy(mean_h, mean_v)

    def issue(ci, p):
        (uid_v, iid_v, fidx_v, UE_v, VA_v, MF_v, LAT_v,
         FU_v, FI_v, UB_v, IB_v) = bufsets[p]
        semA, semB = sems[p]
        cb = base + ci * CHUNK
        pltpu.sync_copy(uid_h.at[pl.ds(cb, CHUNK)], uid_v)
        pltpu.sync_copy(iid_h.at[pl.ds(cb, CHUNK)], iid_v)
        for f in range(N_FEAT):
            pltpu.sync_copy(uft_h.at[pl.ds(f * B + cb, CHUNK)], fidx_v.at[f])

        cpsA = [
            pltpu.async_copy(ue_h.at[uid_v], UE_v, semA),
            pltpu.async_copy(va_h.at[uid_v], VA_v, semA),
            pltpu.async_copy(mf_h.at[iid_v], MF_v, semA),
            pltpu.async_copy(lat_h.at[iid_v], LAT_v, semA),
        ]
        cpsB = [
            pltpu.async_copy(ub_h.at[uid_v], UB_v, semB),
            pltpu.async_copy(ib_h.at[iid_v], IB_v, semB),
        ]
        for f in range(N_FEAT):
            cpsB.append(pltpu.async_copy(fu_h.at[f].at[fidx_v.at[f]],
                                         FU_v.at[f], semB))
            cpsB.append(pltpu.async_copy(fi_h.at[f].at[iid_v],
                                         FI_v.at[f], semB))
        return cpsA, cpsB

    def compute(ci, p, wait_feat):
        (uid_v, iid_v, fidx_v, UE_v, VA_v, MF_v, LAT_v,
         FU_v, FI_v, UB_v, IB_v) = bufsets[p]
        cb = base + ci * CHUNK

        def row_body1(r, _):
            acc = UE_v[r, pl.ds(0, L)] * MF_v[r, pl.ds(0, L)]
            for k in range(1, EMB // L):
                acc += UE_v[r, pl.ds(k * L, L)] * MF_v[r, pl.ds(k * L, L)]
            for k in range(LATENT // L):
                acc += VA_v[r, pl.ds(k * L, L)] * LAT_v[r, pl.ds(k * L, L)]
            part_v[pl.ds(r * L, L)] = acc
            return 0

        lax.fori_loop(0, CHUNK, row_body1, 0)

        wait_feat()

        def row_body2(r, _):
            acc = part_v[pl.ds(r * L, L)]
            for f in range(N_FEAT):
                for k in range(FEAT_DIM // L):
                    acc += (FU_v[f, r, pl.ds(k * L, L)]
                            * FI_v[f, r, pl.ds(k * L, L)])
            part_v[pl.ds(r * L, L)] = acc
            return 0

        lax.fori_loop(0, CHUNK, row_body2, 0)

        def bias_body(g, _):
            biasout_v[pl.ds(g * L, L)] = (mean_v[...]
                                          + UB_v[pl.ds(g * L, L)]
                                          + IB_v[pl.ds(g * L, L)])
            return 0

        lax.fori_loop(0, GROUPS, bias_body, 0)

        pltpu.sync_copy(part_v, part_h.at[pl.ds(cb * L, CHUNK * L)])
        pltpu.sync_copy(biasout_v, bias_h.at[pl.ds(cb, CHUNK)])

    def chunk_body(ci, _):
        cpsA, cpsB = issue(ci, 0)
        for cp in cpsA:
            cp.wait()

        def wait_feat():
            for cp in cpsB:
                cp.wait()

        compute(ci, 0, wait_feat)
        return 0

    lax.fori_loop(0, NCHUNK, chunk_body, 0)


TC_BLK = 2048


def _tc_body(p_ref, b_ref, o_ref):
    o_ref[...] = jnp.sum(p_ref[...], axis=1) + b_ref[...]


@jax.jit
def _call(uid, iid, ufeat_t, ue, ub, mf, ib, fu, fi, mean16, va, lat):
    mesh = plsc.VectorSubcoreMesh(core_axis_name="c", subcore_axis_name="s",
                                  num_cores=NC, num_subcores=NS)
    sc = pl.kernel(
        _sc_body,
        out_type=[jax.ShapeDtypeStruct((B * L,), jnp.float32),
                  jax.ShapeDtypeStruct((B,), jnp.float32)],
        mesh=mesh,
        compiler_params=pltpu.CompilerParams(use_tc_tiling_on_sc=False),
        scratch_types=_SET_TYPES + [
            pltpu.VMEM((L,), jnp.float32),
            pltpu.VMEM((CHUNK * L,), jnp.float32),
            pltpu.VMEM((CHUNK,), jnp.float32),
            pltpu.SemaphoreType.DMA,
            pltpu.SemaphoreType.DMA,
        ],
    )
    partials, biases = sc(uid, iid, ufeat_t, ue, ub, mf, ib, fu, fi,
                          mean16, va, lat)
    out = pl.pallas_call(
        _tc_body,
        out_shape=jax.ShapeDtypeStruct((B,), jnp.float32),
        grid=(B // TC_BLK,),
        in_specs=[
            pl.BlockSpec((TC_BLK, L), lambda i: (i, 0)),
            pl.BlockSpec((TC_BLK,), lambda i: (i,)),
        ],
        out_specs=pl.BlockSpec((TC_BLK,), lambda i: (i,)),
    )(partials.reshape(B, L), biases)
    return out


def kernel(uid, iid, u_feat, user_emb, user_bias, item_emb_mf, item_bias,
           feat_u, feat_i, mean, vae_mean, item_emb_lat):
    return _call(uid, iid, u_feat.T.reshape(-1), user_emb,
                 user_bias.reshape(-1), item_emb_mf, item_bias.reshape(-1),
                 feat_u, feat_i,
                 jnp.broadcast_to(mean, (L,)), vae_mean, item_emb_lat)
